# baseline (speedup 1.0000x reference)
import functools

import jax
import jax.numpy as jnp
from jax import lax
from jax.experimental import pallas as pl
from jax.experimental.pallas import tpu as pltpu
from jax.experimental.pallas import tpu_sc as plsc

_R, _D, _N = 100000, 64, 16384
_NC, _NS = 2, 16
_CS = 16
_SLABS = _D // (_NC * _CS)
_BPT = _N // _NS
_RPT = _R // _NS
_IDX_ROWS = _BPT // 128


def _scatter_add_body(idx_hbm, a_hbm, b_hbm, out_hbm, idx_v, b_v, acc):
    c = lax.axis_index("c")
    s = lax.axis_index("s")
    base = s * _BPT
    pltpu.sync_copy(idx_hbm.at[pl.ds(s * _IDX_ROWS, _IDX_ROWS)], idx_v)
    for k in range(_SLABS):
        col0 = (c * _SLABS + k) * _CS
        pltpu.sync_copy(
            a_hbm.at[pl.ds(s * _RPT, _RPT), pl.ds(col0, _CS)],
            acc.at[pl.ds(s * _RPT, _RPT), :],
        )
        pltpu.sync_copy(b_hbm.at[pl.ds(base, _BPT), pl.ds(col0, _CS)], b_v)
        plsc.subcore_barrier()
        for j in range(_IDX_ROWS):
            pltpu.sync_copy(
                b_v.at[pl.ds(j * 128, 128), :],
                acc.at[idx_v.at[j]],
                add=True,
            )
        plsc.subcore_barrier()
        pltpu.sync_copy(
            acc.at[pl.ds(s * _RPT, _RPT), :],
            out_hbm.at[pl.ds(s * _RPT, _RPT), pl.ds(col0, _CS)],
        )
        if k + 1 < _SLABS:
            plsc.subcore_barrier()


_sc_scatter_add = functools.partial(
    pl.kernel,
    out_type=jax.ShapeDtypeStruct((_R, _D), jnp.float32),
    mesh=plsc.VectorSubcoreMesh(
        core_axis_name="c", subcore_axis_name="s",
        num_cores=_NC, num_subcores=_NS,
    ),
    scratch_types=[
        pltpu.VMEM((_NS * _IDX_ROWS // _NS, 128), jnp.int32),
        pltpu.VMEM((_BPT, _CS), jnp.float32),
        pltpu.VMEM_SHARED((_R, _CS), jnp.float32),
    ],
    compiler_params=pltpu.CompilerParams(use_tc_tiling_on_sc=False),
)(_scatter_add_body)


@jax.jit
def kernel(index, A, B):
    idx2d = index.astype(jnp.int32).reshape(_NS * _IDX_ROWS, 128)
    return _sc_scatter_add(idx2d, A, B)
